# baseline (device time: 45104 ns/iter reference)
import jax
import jax.numpy as jnp
from jax import lax
from jax.experimental import pallas as pl
from jax.experimental.pallas import tpu as pltpu

N_DEV = 8
B = 2
S = 256
D_MODEL = 512
HPB = 4
DH = 64
HB = HPB * DH

L_HOPS = N_DEV // 2
R_HOPS = N_DEV - 1 - L_HOPS


def kernel(x, Wq, K_ext, V_ext, Wo):
    K_t = jnp.transpose(K_ext, (0, 2, 1, 3)).astype(jnp.bfloat16)
    V_t = jnp.transpose(V_ext, (0, 2, 1, 3)).astype(jnp.bfloat16)
    x_b = x.astype(jnp.bfloat16)
    blk = jnp.concatenate(
        [Wq.T.astype(jnp.bfloat16), Wo.astype(jnp.bfloat16)], axis=0)

    def body(x_ref, blk_ref, k_ref, v_ref, out_ref, comm, ssem, rsem):
        def sigma(v):
            return jnp.where(v < 4, v, 11 - v)

        my_pos = lax.axis_index("i")
        vi = sigma(my_pos)
        left = sigma(lax.rem(vi + N_DEV - 1, N_DEV))
        right = sigma(lax.rem(vi + 1, N_DEV))

        barrier_sem = pltpu.get_barrier_semaphore()
        for nbr in (left, right):
            pl.semaphore_signal(
                barrier_sem, inc=1,
                device_id=(nbr,), device_id_type=pl.DeviceIdType.MESH,
            )
        pl.semaphore_wait(barrier_sem, 2)

        comm[0] = blk_ref[...]

        def origin_of(slot):
            if slot == 0:
                return my_pos
            if slot <= R_HOPS:
                return sigma(lax.rem(vi - slot + N_DEV, N_DEV))
            return sigma(lax.rem(vi + (slot - R_HOPS), N_DEV))

        def compute(slot):
            head0 = origin_of(slot) * HPB
            wqT = comm[slot, :HB, :]
            wo = comm[slot, HB:, :]
            for b in range(B):
                q = lax.dot_general(
                    x_ref[b], wqT, (((1,), (1,)), ((), ())),
                    preferred_element_type=jnp.float32)
                kblk4 = k_ref[b, pl.ds(head0, HPB)]
                vblk4 = v_ref[b, pl.ds(head0, HPB)]
                ctx_parts = []
                for h in range(HPB):
                    qh = (q[:, h * DH:(h + 1) * DH]
                          .astype(jnp.bfloat16).reshape(4, 64, DH))
                    kh = kblk4[h].reshape(4, 64, DH)
                    vh = vblk4[h].reshape(4, 64, DH)
                    scores = lax.dot_general(
                        qh, kh, (((2,), (2,)), ((0,), (0,))),
                        preferred_element_type=jnp.float32,
                    ) * 0.125
                    e = jnp.exp(scores)
                    w = (e / jnp.sum(e, axis=2, keepdims=True)
                         ).astype(jnp.bfloat16)
                    ctx_parts.append(
                        lax.dot_general(
                            w, vh, (((2,), (1,)), ((0,), (0,))),
                            preferred_element_type=jnp.float32,
                        ).reshape(S, DH))
                ctx = jnp.concatenate(ctx_parts, axis=1)
                contrib = jnp.dot(ctx.astype(jnp.bfloat16), wo,
                                  preferred_element_type=jnp.float32)
                if slot == 0:
                    out_ref[b] = contrib
                else:
                    out_ref[b] = out_ref[b] + contrib

        def fwd(src_slot, dst_slot, dev):
            r = pltpu.make_async_remote_copy(
                src_ref=comm.at[src_slot], dst_ref=comm.at[dst_slot],
                send_sem=ssem.at[dst_slot], recv_sem=rsem.at[dst_slot],
                device_id=(dev,), device_id_type=pl.DeviceIdType.MESH,
            )
            r.start()
            return r

        for t in range(1, L_HOPS + 1):
            started = []
            if t <= R_HOPS:
                started.append(fwd(t - 1, t, right))
            lsrc = 0 if t == 1 else R_HOPS + (t - 1)
            started.append(fwd(lsrc, R_HOPS + t, left))
            if t == 1:
                compute(0)
            else:
                compute(t - 1)
                compute(R_HOPS + (t - 1))
            for r in started:
                r.wait()
        compute(R_HOPS + L_HOPS)

    return pl.pallas_call(
        body,
        out_shape=jax.ShapeDtypeStruct((B, S, D_MODEL), jnp.float32),
        in_specs=[pl.BlockSpec(memory_space=pltpu.VMEM)] * 4,
        out_specs=pl.BlockSpec(memory_space=pltpu.VMEM),
        scratch_shapes=[
            pltpu.VMEM((N_DEV, 2 * HB, D_MODEL), jnp.bfloat16),
            pltpu.SemaphoreType.DMA((N_DEV,)),
            pltpu.SemaphoreType.DMA((N_DEV,)),
        ],
        compiler_params=pltpu.CompilerParams(collective_id=0),
    )(x_b, blk, K_t, V_t)
